# baseline (speedup 1.0000x reference)
import functools

import jax
import jax.numpy as jnp
from jax import lax
from jax.experimental import pallas as pl
from jax.experimental.pallas import tpu as pltpu
from jax.experimental.pallas import tpu_sc as plsc

_STAGE_REPEATS = 32


def kernel(genre, genre_embed_weight):
    bs, num_embed = genre.shape
    embed_dim = genre_embed_weight.shape[1]
    mesh = plsc.VectorSubcoreMesh(core_axis_name="c", subcore_axis_name="s")
    num_cores, num_sub = mesh.num_cores, mesh.num_subcores
    stage_cols = _STAGE_REPEATS * num_embed
    n_bands = bs // _STAGE_REPEATS
    bands_per_sub = n_bands // (num_cores * num_sub)

    @functools.partial(
        pl.kernel,
        out_type=jax.ShapeDtypeStruct(
            (embed_dim, bs * num_embed), genre_embed_weight.dtype
        ),
        mesh=mesh,
        scratch_types=[
            pltpu.VMEM_SHARED((embed_dim, stage_cols), genre_embed_weight.dtype),
            pltpu.SemaphoreType.DMA,
        ],
    )
    def band_fill(wt_hbm, out_hbm, stage, sem):
        c = lax.axis_index("c")
        s = lax.axis_index("s")

        @pl.when(s == 0)
        def _fill():
            pltpu.sync_copy(wt_hbm, stage)

        plsc.subcore_barrier()
        base = (c * num_sub + s) * bands_per_sub
        copies = [
            pltpu.make_async_copy(
                stage,
                out_hbm.at[:, pl.ds((base + k) * stage_cols, stage_cols)],
                sem,
            )
            for k in range(bands_per_sub)
        ]
        for cp in copies:
            cp.start()
        for cp in copies:
            cp.wait()

    wt_band = jnp.tile(genre_embed_weight.T, (1, _STAGE_REPEATS))
    out2d = band_fill(wt_band)
    return out2d.T[:, None, :]


# speedup vs baseline: 1.0420x; 1.0420x over previous
import functools

import jax
import jax.numpy as jnp
from jax import lax
from jax.experimental import pallas as pl
from jax.experimental.pallas import tpu as pltpu
from jax.experimental.pallas import tpu_sc as plsc

_STAGE_REPEATS = 16


def kernel(genre, genre_embed_weight):
    bs, num_embed = genre.shape
    embed_dim = genre_embed_weight.shape[1]
    mesh = plsc.VectorSubcoreMesh(core_axis_name="c", subcore_axis_name="s")
    num_cores, num_sub = mesh.num_cores, mesh.num_subcores
    stage_cols = _STAGE_REPEATS * num_embed
    n_bands = bs // _STAGE_REPEATS
    bands_per_sub = n_bands // (num_cores * num_sub)

    @functools.partial(
        pl.kernel,
        out_type=jax.ShapeDtypeStruct(
            (embed_dim, bs * num_embed), genre_embed_weight.dtype
        ),
        mesh=mesh,
        scratch_types=[
            pltpu.VMEM_SHARED((embed_dim, stage_cols), genre_embed_weight.dtype),
            pltpu.SemaphoreType.DMA,
        ],
    )
    def band_fill(wt_hbm, out_hbm, stage, sem):
        c = lax.axis_index("c")
        s = lax.axis_index("s")

        @pl.when(s == 0)
        def _fill():
            pltpu.sync_copy(wt_hbm, stage)

        plsc.subcore_barrier()
        base = (c * num_sub + s) * bands_per_sub
        copies = [
            pltpu.make_async_copy(
                stage,
                out_hbm.at[:, pl.ds((base + k) * stage_cols, stage_cols)],
                sem,
            )
            for k in range(bands_per_sub)
        ]
        for cp in copies:
            cp.start()
        for cp in copies:
            cp.wait()

    wt_band = jnp.tile(genre_embed_weight.T, (1, _STAGE_REPEATS))
    out2d = band_fill(wt_band)
    return out2d.T[:, None, :]
